# baseline (device time: 12085 ns/iter reference)
import jax
import jax.numpy as jnp
from jax import lax
from jax.experimental import pallas as pl
from jax.experimental.pallas import tpu as pltpu

N_DEV = 4
N_TOK = 512
D_IN = 256
D_OUT = 512
E_PER = 2
CAP = 51
CHUNK = N_TOK // N_DEV


def kernel(x, router_W, route_idx, expert_W):
    del router_W

    def body(x_ref, route_ref, ew_ref, out_ref, keep_ref, ewb_ref,
             send_ref, recv_ref, send_sems, recv_sems):
        my = lax.axis_index("i")

        barrier_sem = pltpu.get_barrier_semaphore()
        for off in range(1, N_DEV):
            pl.semaphore_signal(
                barrier_sem, inc=1,
                device_id=(lax.rem(my + off, N_DEV),),
                device_id_type=pl.DeviceIdType.MESH,
            )

        route = route_ref[:, :]
        e_ids = my * E_PER + jnp.arange(E_PER, dtype=jnp.int32)
        onehot = route == e_ids[None, :]
        row = lax.broadcasted_iota(jnp.int32, (N_TOK, N_TOK), 0)
        col = lax.broadcasted_iota(jnp.int32, (N_TOK, N_TOK), 1)
        tri = (col <= row).astype(jnp.bfloat16)
        cnt = jnp.dot(tri, onehot.astype(jnp.bfloat16),
                      preferred_element_type=jnp.float32)
        keep_ref[:, :] = jnp.logical_and(onehot, cnt <= CAP).astype(
            jnp.bfloat16)
        ewb_ref[:, :, :] = ew_ref[:, :, :].astype(jnp.bfloat16)

        def chunk_partial(c):
            xc = x_ref[pl.ds(c * CHUNK, CHUNK), :].astype(jnp.bfloat16)
            kc = keep_ref[pl.ds(c * CHUNK, CHUNK), :]
            p = jnp.dot(xc * kc[:, 0:1], ewb_ref[0],
                        preferred_element_type=jnp.float32)
            p += jnp.dot(xc * kc[:, 1:2], ewb_ref[1],
                         preferred_element_type=jnp.float32)
            return p

        rdmas = []
        for k, off in enumerate((2, 1, 3)):
            dest = lax.rem(my + off, N_DEV)
            send_ref[k, :, :] = chunk_partial(dest).astype(jnp.bfloat16)
            if k == 0:
                pl.semaphore_wait(barrier_sem, N_DEV - 1)
            rdma = pltpu.make_async_remote_copy(
                src_ref=send_ref.at[k],
                dst_ref=recv_ref.at[3 - off],
                send_sem=send_sems.at[k],
                recv_sem=recv_sems.at[3 - off],
                device_id=(dest,),
                device_id_type=pl.DeviceIdType.MESH,
            )
            rdma.start()
            rdmas.append(rdma)

        own = chunk_partial(my)

        for rdma in rdmas:
            rdma.wait_recv()
        out_ref[:, :] = own + (
            recv_ref[0, :, :] + recv_ref[1, :, :] + recv_ref[2, :, :]
        ).astype(jnp.float32)

        for rdma in rdmas:
            rdma.wait_send()

    return pl.pallas_call(
        body,
        out_shape=jax.ShapeDtypeStruct((CHUNK, D_OUT), jnp.float32),
        in_specs=[
            pl.BlockSpec(memory_space=pltpu.VMEM),
            pl.BlockSpec(memory_space=pltpu.VMEM),
            pl.BlockSpec(memory_space=pltpu.VMEM),
        ],
        out_specs=pl.BlockSpec(memory_space=pltpu.VMEM),
        scratch_shapes=[
            pltpu.VMEM((N_TOK, E_PER), jnp.bfloat16),
            pltpu.VMEM((E_PER, D_IN, D_OUT), jnp.bfloat16),
            pltpu.VMEM((N_DEV - 1, CHUNK, D_OUT), jnp.bfloat16),
            pltpu.VMEM((N_DEV - 1, CHUNK, D_OUT), jnp.bfloat16),
            pltpu.SemaphoreType.DMA((N_DEV - 1,)),
            pltpu.SemaphoreType.DMA((N_DEV - 1,)),
        ],
        compiler_params=pltpu.CompilerParams(collective_id=0),
    )(x, route_idx, expert_W)


# device time: 11025 ns/iter; 1.0961x vs baseline; 1.0961x over previous
import jax
import jax.numpy as jnp
from jax import lax
from jax.experimental import pallas as pl
from jax.experimental.pallas import tpu as pltpu

N_DEV = 4
N_TOK = 512
D_IN = 256
D_OUT = 512
E_PER = 2
CAP = 51
CHUNK = N_TOK // N_DEV


def kernel(x, router_W, route_idx, expert_W):
    del router_W

    x_bf = x.astype(jnp.bfloat16)
    ew_bf = expert_W.astype(jnp.bfloat16)

    def body(x_ref, route_ref, ew_ref, out_ref, keep_ref,
             send_ref, recv_ref, send_sems, recv_sems):
        my = lax.axis_index("i")

        barrier_sem = pltpu.get_barrier_semaphore()
        for off in range(1, N_DEV):
            pl.semaphore_signal(
                barrier_sem, inc=1,
                device_id=(lax.rem(my + off, N_DEV),),
                device_id_type=pl.DeviceIdType.MESH,
            )

        route = route_ref[:, :]
        e_ids = my * E_PER + jnp.arange(E_PER, dtype=jnp.int32)
        onehot = route == e_ids[None, :]
        row = lax.broadcasted_iota(jnp.int32, (N_TOK, N_TOK), 0)
        col = lax.broadcasted_iota(jnp.int32, (N_TOK, N_TOK), 1)
        tri = (col <= row).astype(jnp.bfloat16)
        cnt = jnp.dot(tri, onehot.astype(jnp.bfloat16),
                      preferred_element_type=jnp.float32)
        keep_ref[:, :] = jnp.logical_and(onehot, cnt <= CAP).astype(
            jnp.bfloat16)

        def chunk_partial(c):
            xc = x_ref[pl.ds(c * CHUNK, CHUNK), :]
            kc = keep_ref[pl.ds(c * CHUNK, CHUNK), :]
            p = jnp.dot(xc * kc[:, 0:1], ew_ref[0],
                        preferred_element_type=jnp.float32)
            p += jnp.dot(xc * kc[:, 1:2], ew_ref[1],
                         preferred_element_type=jnp.float32)
            return p

        rdmas = []
        for k, off in enumerate((2, 1, 3)):
            dest = lax.rem(my + off, N_DEV)
            send_ref[k, :, :] = chunk_partial(dest).astype(jnp.bfloat16)
            if k == 0:
                pl.semaphore_wait(barrier_sem, N_DEV - 1)
            rdma = pltpu.make_async_remote_copy(
                src_ref=send_ref.at[k],
                dst_ref=recv_ref.at[3 - off],
                send_sem=send_sems.at[k],
                recv_sem=recv_sems.at[3 - off],
                device_id=(dest,),
                device_id_type=pl.DeviceIdType.MESH,
            )
            rdma.start()
            rdmas.append(rdma)

        own = chunk_partial(my).astype(jnp.bfloat16)

        for rdma in rdmas:
            rdma.wait_recv()
        out_ref[:, :] = own + (
            recv_ref[0, :, :] + recv_ref[1, :, :] + recv_ref[2, :, :]
        )

        for rdma in rdmas:
            rdma.wait_send()

    return pl.pallas_call(
        body,
        out_shape=jax.ShapeDtypeStruct((CHUNK, D_OUT), jnp.bfloat16),
        in_specs=[
            pl.BlockSpec(memory_space=pltpu.VMEM),
            pl.BlockSpec(memory_space=pltpu.VMEM),
            pl.BlockSpec(memory_space=pltpu.VMEM),
        ],
        out_specs=pl.BlockSpec(memory_space=pltpu.VMEM),
        scratch_shapes=[
            pltpu.VMEM((N_TOK, E_PER), jnp.bfloat16),
            pltpu.VMEM((N_DEV - 1, CHUNK, D_OUT), jnp.bfloat16),
            pltpu.VMEM((N_DEV - 1, CHUNK, D_OUT), jnp.bfloat16),
            pltpu.SemaphoreType.DMA((N_DEV - 1,)),
            pltpu.SemaphoreType.DMA((N_DEV - 1,)),
        ],
        compiler_params=pltpu.CompilerParams(collective_id=0),
    )(x_bf, route_idx, ew_bf)


# device time: 10505 ns/iter; 1.1504x vs baseline; 1.0495x over previous
import jax
import jax.numpy as jnp
from jax import lax
from jax.experimental import pallas as pl
from jax.experimental.pallas import tpu as pltpu

N_DEV = 4
N_TOK = 512
D_IN = 256
D_OUT = 512
E_PER = 2
CAP = 51
CHUNK = N_TOK // N_DEV


def kernel(x, router_W, route_idx, expert_W):
    del router_W

    x_bf = x.astype(jnp.bfloat16)
    ew_bf = expert_W.astype(jnp.bfloat16)

    def body(x_ref, route_ref, ew_ref, out_ref, keep_ref,
             send_ref, recv_ref, send_sems, recv_sems):
        my = lax.axis_index("i")

        barrier_sem = pltpu.get_barrier_semaphore()
        for off in range(1, N_DEV):
            pl.semaphore_signal(
                barrier_sem, inc=1,
                device_id=(lax.rem(my + off, N_DEV),),
                device_id_type=pl.DeviceIdType.MESH,
            )

        route = route_ref[:, :]
        e_ids = my * E_PER + jnp.arange(E_PER, dtype=jnp.int32)
        onehot = route == e_ids[None, :]
        row = lax.broadcasted_iota(jnp.int32, (N_TOK, N_TOK), 0)
        col = lax.broadcasted_iota(jnp.int32, (N_TOK, N_TOK), 1)
        tri = (col <= row).astype(jnp.bfloat16)
        cnt = jnp.dot(tri, onehot.astype(jnp.bfloat16),
                      preferred_element_type=jnp.float32)
        keep_ref[:, :] = jnp.logical_and(onehot, cnt <= CAP).astype(
            jnp.bfloat16)

        def chunk_partial(c):
            xc = x_ref[pl.ds(c * CHUNK, CHUNK), :]
            kc = keep_ref[pl.ds(c * CHUNK, CHUNK), :]
            p = jnp.dot(xc * kc[:, 0:1], ew_ref[0],
                        preferred_element_type=jnp.float32)
            p += jnp.dot(xc * kc[:, 1:2], ew_ref[1],
                         preferred_element_type=jnp.float32)
            return p

        offs = (2, 1, 3)
        for k, off in enumerate(offs):
            dest = lax.rem(my + off, N_DEV)
            send_ref[k, :, :] = chunk_partial(dest).astype(jnp.bfloat16)
        own = chunk_partial(my).astype(jnp.bfloat16)

        pl.semaphore_wait(barrier_sem, N_DEV - 1)
        rdmas = []
        for k, off in enumerate(offs):
            rdma = pltpu.make_async_remote_copy(
                src_ref=send_ref.at[k],
                dst_ref=recv_ref.at[3 - off],
                send_sem=send_sems.at[k],
                recv_sem=recv_sems.at[3 - off],
                device_id=(lax.rem(my + off, N_DEV),),
                device_id_type=pl.DeviceIdType.MESH,
            )
            rdma.start()
            rdmas.append(rdma)

        for rdma in rdmas:
            rdma.wait_recv()
        out_ref[:, :] = own + (
            recv_ref[0, :, :] + recv_ref[1, :, :] + recv_ref[2, :, :]
        )

        for rdma in rdmas:
            rdma.wait_send()

    return pl.pallas_call(
        body,
        out_shape=jax.ShapeDtypeStruct((CHUNK, D_OUT), jnp.bfloat16),
        in_specs=[
            pl.BlockSpec(memory_space=pltpu.VMEM),
            pl.BlockSpec(memory_space=pltpu.VMEM),
            pl.BlockSpec(memory_space=pltpu.VMEM),
        ],
        out_specs=pl.BlockSpec(memory_space=pltpu.VMEM),
        scratch_shapes=[
            pltpu.VMEM((N_TOK, E_PER), jnp.bfloat16),
            pltpu.VMEM((N_DEV - 1, CHUNK, D_OUT), jnp.bfloat16),
            pltpu.VMEM((N_DEV - 1, CHUNK, D_OUT), jnp.bfloat16),
            pltpu.SemaphoreType.DMA((N_DEV - 1,)),
            pltpu.SemaphoreType.DMA((N_DEV - 1,)),
        ],
        compiler_params=pltpu.CompilerParams(collective_id=0),
    )(x_bf, route_idx, ew_bf)
